# Initial kernel scaffold; baseline (speedup 1.0000x reference)
import jax
import jax.numpy as jnp
from jax.experimental import pallas as pl


def kernel(x, edge_index, W, b, a, u):
    raise NotImplementedError("write your pallas kernel here")


# speedup vs baseline: 17.6483x; 17.6483x over previous
import functools

import jax
import jax.numpy as jnp
from jax import lax
from jax.experimental import pallas as pl
from jax.experimental.pallas import tpu as pltpu
from jax.experimental.pallas import tpu_sc as plsc

N = 10000
E = 320000
F = 128
NC = 2
NS = 16
NW = NC * NS
EPW = E // NW
CH = 80
NCHUNK = EPW // CH
NP = NS * 640
BN = 1000
GRID = N // BN

_mesh = plsc.VectorSubcoreMesh(core_axis_name="c", subcore_axis_name="s")


@functools.partial(
    pl.kernel,
    out_type=jax.ShapeDtypeStruct((NC, NP), jnp.float32),
    mesh=_mesh,
    scratch_types=[
        pltpu.VMEM((1, CH), jnp.int32),
        pltpu.VMEM((CH,), jnp.float32),
        pltpu.VMEM((640,), jnp.float32),
        pltpu.VMEM_SHARED((NP,), jnp.float32),
    ],
)
def _deg_kernel(dst_hbm, zeros_hbm, ones_hbm, hist_hbm, didx, ones_v, zb, hist_sp):
    c = lax.axis_index("c")
    s = lax.axis_index("s")
    pltpu.sync_copy(zeros_hbm, zb)
    pltpu.sync_copy(ones_hbm, ones_v)
    pltpu.sync_copy(zb, hist_sp.at[pl.ds(s * 640, 640)])
    plsc.subcore_barrier()
    wid = c * NS + s

    def body(j, carry):
        off = wid * EPW + j * CH
        pltpu.sync_copy(dst_hbm.at[pl.ds(off, CH)], didx.at[0])
        pltpu.sync_copy(ones_v, hist_sp.at[didx.at[0]], add=True)
        return carry

    lax.fori_loop(0, NCHUNK, body, 0)
    plsc.subcore_barrier()
    pltpu.sync_copy(hist_sp.at[pl.ds(s * 640, 640)], zb)
    pltpu.sync_copy(zb, hist_hbm.at[c, pl.ds(s * 640, 640)])


def _mm_body(hist_ref, x_ref, w_ref, u_ref, h2_ref, dis_ref, wsn_ref):
    @pl.when(pl.program_id(0) == 0)
    def _():
        W = w_ref[...]
        u_row = u_ref[...]
        v = lax.dot_general(u_row, W, (((1,), (0,)), ((), ())))
        v = v / (jnp.sqrt(jnp.sum(v * v)) + 1e-12)
        wv = lax.dot_general(v, W, (((1,), (1,)), ((), ())))
        u2 = wv / (jnp.sqrt(jnp.sum(wv * wv)) + 1e-12)
        sigma = jnp.sum(u2 * wv)
        wsn_ref[...] = W / sigma

    ht = hist_ref[...]
    deg = ht[:, 0:1] + ht[:, 1:2] + 1.0
    dis = lax.rsqrt(deg)
    dis_ref[...] = dis
    h = lax.dot_general(x_ref[...], wsn_ref[...], (((1,), (1,)), ((), ())),
                        preferred_element_type=jnp.float32)
    h2_ref[...] = h * dis


_mm_call = pl.pallas_call(
    _mm_body,
    grid=(GRID,),
    in_specs=[
        pl.BlockSpec((BN, NC), lambda i: (i, 0)),
        pl.BlockSpec((BN, F), lambda i: (i, 0)),
        pl.BlockSpec((F, F), lambda i: (0, 0)),
        pl.BlockSpec((1, F), lambda i: (0, 0)),
    ],
    out_specs=[
        pl.BlockSpec((BN, F), lambda i: (i, 0)),
        pl.BlockSpec((BN, 1), lambda i: (i, 0)),
    ],
    out_shape=[
        jax.ShapeDtypeStruct((N, F), jnp.float32),
        jax.ShapeDtypeStruct((N, 1), jnp.float32),
    ],
    scratch_shapes=[pltpu.VMEM((F, F), jnp.float32)],
)


@functools.partial(
    pl.kernel,
    out_type=jax.ShapeDtypeStruct((NC, NP, F), jnp.float32),
    mesh=_mesh,
    scratch_types=[
        pltpu.VMEM((CH,), jnp.int32),
        pltpu.VMEM((1, CH), jnp.int32),
        pltpu.VMEM((CH, F), jnp.float32),
        pltpu.VMEM((CH, F), jnp.float32),
        pltpu.VMEM_SHARED((NP, F), jnp.float32),
        pltpu.SemaphoreType.DMA,
    ],
)
def _scatter_kernel(h2_hbm, src_hbm, dst_hbm, zrows_hbm, part_hbm,
                    sidx, didx, rows, zrows, acc_sp, sem):
    c = lax.axis_index("c")
    s = lax.axis_index("s")
    pltpu.sync_copy(zrows_hbm, zrows)
    for k in range(640 // CH):
        pltpu.sync_copy(zrows, acc_sp.at[pl.ds(s * 640 + k * CH, CH)])
    plsc.subcore_barrier()
    wid = c * NS + s

    def body(j, carry):
        off = wid * EPW + j * CH
        pltpu.sync_copy(src_hbm.at[pl.ds(off, CH)], sidx)
        pltpu.sync_copy(dst_hbm.at[pl.ds(off, CH)], didx.at[0])
        pltpu.async_copy(h2_hbm.at[sidx], rows, sem).wait()
        pltpu.sync_copy(rows, acc_sp.at[didx.at[0]], add=True)
        return carry

    lax.fori_loop(0, NCHUNK, body, 0)
    plsc.subcore_barrier()
    for k in range(640 // CH):
        pltpu.sync_copy(acc_sp.at[pl.ds(s * 640 + k * CH, CH)], rows)
        pltpu.sync_copy(rows, part_hbm.at[c, pl.ds(s * 640 + k * CH, CH)])


def _ep_body(p_ref, h2_ref, dis_ref, b_ref, a_ref, out_ref):
    z = (p_ref[0] + p_ref[1] + h2_ref[...]) * dis_ref[...] + b_ref[...]
    out_ref[...] = jnp.where(z > 0, z, a_ref[0, 0] * z)


_ep_call = pl.pallas_call(
    _ep_body,
    grid=(GRID,),
    in_specs=[
        pl.BlockSpec((NC, BN, F), lambda i: (0, i, 0)),
        pl.BlockSpec((BN, F), lambda i: (i, 0)),
        pl.BlockSpec((BN, 1), lambda i: (i, 0)),
        pl.BlockSpec((1, F), lambda i: (0, 0)),
        pl.BlockSpec((1, 1), lambda i: (0, 0)),
    ],
    out_specs=pl.BlockSpec((BN, F), lambda i: (i, 0)),
    out_shape=jax.ShapeDtypeStruct((N, F), jnp.float32),
)


def kernel(x, edge_index, W, b, a, u):
    src = edge_index[0].astype(jnp.int32)
    dst = edge_index[1].astype(jnp.int32)
    zeros640 = jnp.zeros((640,), jnp.float32)
    ones_ch = jnp.ones((CH,), jnp.float32)
    zrows = jnp.zeros((CH, F), jnp.float32)

    hist = _deg_kernel(dst, zeros640, ones_ch)
    h2, dis = _mm_call(hist.T, x, W, u.reshape(1, F))
    part = _scatter_kernel(h2, src, dst, zrows)
    return _ep_call(part, h2, dis, b.reshape(1, F), a.reshape(1, 1))


# speedup vs baseline: 39.9934x; 2.2661x over previous
import functools

import jax
import jax.numpy as jnp
from jax import lax
from jax.experimental import pallas as pl
from jax.experimental.pallas import tpu as pltpu
from jax.experimental.pallas import tpu_sc as plsc

N = 10000
E = 320000
F = 128
NC = 2
NS = 16
NW = NC * NS
EPW = E // NW
CH = 80
NCHUNK = EPW // CH
NP = NS * 640
BN = 1000
GRID = N // BN

_mesh = plsc.VectorSubcoreMesh(core_axis_name="c", subcore_axis_name="s")


@functools.partial(
    pl.kernel,
    out_type=jax.ShapeDtypeStruct((NC, NP), jnp.float32),
    mesh=_mesh,
    scratch_types=[
        pltpu.VMEM((NCHUNK, CH), jnp.int32),
        pltpu.VMEM((CH,), jnp.float32),
        pltpu.VMEM((640,), jnp.float32),
        pltpu.VMEM_SHARED((NP,), jnp.float32),
        pltpu.SemaphoreType.DMA,
        pltpu.SemaphoreType.DMA,
    ],
)
def _deg_kernel(dst3_hbm, zeros_hbm, ones_hbm, hist_hbm, didx_all, ones_v, zb,
                hist_sp, sem0, sem1):
    c = lax.axis_index("c")
    s = lax.axis_index("s")
    pltpu.sync_copy(zeros_hbm, zb)
    pltpu.sync_copy(ones_hbm, ones_v)
    pltpu.sync_copy(zb, hist_sp.at[pl.ds(s * 640, 640)])
    wid = c * NS + s
    pltpu.sync_copy(dst3_hbm.at[wid], didx_all)
    plsc.subcore_barrier()

    def addchunk(j, sem):
        pltpu.async_copy(ones_v, hist_sp.at[didx_all.at[j]], sem, add=True)

    def drain(sem):
        pltpu.make_async_copy(zeros_hbm.at[pl.ds(0, CH)], ones_v, sem).wait()

    addchunk(0, sem0)

    def pair(p, carry):
        j0 = 2 * p
        addchunk(j0 + 1, sem1)
        drain(sem0)
        addchunk(j0 + 2, sem0)
        drain(sem1)
        return carry

    lax.fori_loop(0, (NCHUNK - 1) // 2, pair, 0)
    drain(sem0)
    plsc.subcore_barrier()
    pltpu.sync_copy(hist_sp.at[pl.ds(s * 640, 640)], zb)
    pltpu.sync_copy(zb, hist_hbm.at[c, pl.ds(s * 640, 640)])


def _mm_body(x_ref, w_ref, u_ref, h_ref, wsn_ref):
    @pl.when(pl.program_id(0) == 0)
    def _():
        W = w_ref[...]
        u_row = u_ref[...]
        v = lax.dot_general(u_row, W, (((1,), (0,)), ((), ())))
        v = v / (jnp.sqrt(jnp.sum(v * v)) + 1e-12)
        wv = lax.dot_general(v, W, (((1,), (1,)), ((), ())))
        u2 = wv / (jnp.sqrt(jnp.sum(wv * wv)) + 1e-12)
        sigma = jnp.sum(u2 * wv)
        wsn_ref[...] = W / sigma

    h_ref[...] = lax.dot_general(x_ref[...], wsn_ref[...],
                                 (((1,), (1,)), ((), ())),
                                 preferred_element_type=jnp.float32)


_mm_call = pl.pallas_call(
    _mm_body,
    grid=(GRID,),
    in_specs=[
        pl.BlockSpec((BN, F), lambda i: (i, 0)),
        pl.BlockSpec((F, F), lambda i: (0, 0)),
        pl.BlockSpec((1, F), lambda i: (0, 0)),
    ],
    out_specs=pl.BlockSpec((BN, F), lambda i: (i, 0)),
    out_shape=jax.ShapeDtypeStruct((N, F), jnp.float32),
    scratch_shapes=[pltpu.VMEM((F, F), jnp.float32)],
)


def _scale_body(hist_ref, h_ref, h2_ref, dis_ref):
    ht = hist_ref[...]
    deg = ht[:, 0:1] + ht[:, 1:2] + 1.0
    dis = lax.rsqrt(deg)
    dis_ref[...] = dis
    h2_ref[...] = h_ref[...] * dis


_scale_call = pl.pallas_call(
    _scale_body,
    grid=(GRID,),
    in_specs=[
        pl.BlockSpec((BN, NC), lambda i: (i, 0)),
        pl.BlockSpec((BN, F), lambda i: (i, 0)),
    ],
    out_specs=[
        pl.BlockSpec((BN, F), lambda i: (i, 0)),
        pl.BlockSpec((BN, 1), lambda i: (i, 0)),
    ],
    out_shape=[
        jax.ShapeDtypeStruct((N, F), jnp.float32),
        jax.ShapeDtypeStruct((N, 1), jnp.float32),
    ],
)


@functools.partial(
    pl.kernel,
    out_type=jax.ShapeDtypeStruct((NC, NP, F), jnp.float32),
    mesh=_mesh,
    scratch_types=[
        pltpu.VMEM((EPW,), jnp.int32),
        pltpu.VMEM((NCHUNK, CH), jnp.int32),
        pltpu.VMEM((CH, F), jnp.float32),
        pltpu.VMEM((CH, F), jnp.float32),
        pltpu.VMEM_SHARED((NP, F), jnp.float32),
        pltpu.SemaphoreType.DMA,
        pltpu.SemaphoreType.DMA,
    ],
)
def _scatter_kernel(h2_hbm, src2_hbm, dst3_hbm, zrows_hbm, part_hbm,
                    sidx_all, didx_all, rows0, rows1, acc_sp, sem0, sem1):
    c = lax.axis_index("c")
    s = lax.axis_index("s")
    wid = c * NS + s
    pltpu.sync_copy(zrows_hbm, rows0)
    for k in range(640 // CH):
        pltpu.sync_copy(rows0, acc_sp.at[pl.ds(s * 640 + k * CH, CH)])
    pltpu.sync_copy(src2_hbm.at[wid], sidx_all)
    pltpu.sync_copy(dst3_hbm.at[wid], didx_all)
    plsc.subcore_barrier()

    def gather(j, buf, sem):
        pltpu.async_copy(h2_hbm.at[sidx_all.at[pl.ds(j * CH, CH)]], buf, sem)

    def gwait(buf, sem):
        pltpu.make_async_copy(h2_hbm.at[pl.ds(0, CH)], buf, sem).wait()

    gather(0, rows0, sem0)

    def pair(p, carry):
        j0 = 2 * p
        gather(j0 + 1, rows1, sem1)
        gwait(rows0, sem0)
        pltpu.sync_copy(rows0, acc_sp.at[didx_all.at[j0]], add=True)
        gather(j0 + 2, rows0, sem0)
        gwait(rows1, sem1)
        pltpu.sync_copy(rows1, acc_sp.at[didx_all.at[j0 + 1]], add=True)
        return carry

    lax.fori_loop(0, (NCHUNK - 1) // 2, pair, 0)
    gwait(rows0, sem0)
    pltpu.sync_copy(rows0, acc_sp.at[didx_all.at[NCHUNK - 1]], add=True)
    plsc.subcore_barrier()
    for k in range(640 // CH):
        pltpu.sync_copy(acc_sp.at[pl.ds(s * 640 + k * CH, CH)], rows0)
        pltpu.sync_copy(rows0, part_hbm.at[c, pl.ds(s * 640 + k * CH, CH)])


def _ep_body(p_ref, h2_ref, dis_ref, b_ref, a_ref, out_ref):
    z = (p_ref[0] + p_ref[1] + h2_ref[...]) * dis_ref[...] + b_ref[...]
    out_ref[...] = jnp.where(z > 0, z, a_ref[0, 0] * z)


_ep_call = pl.pallas_call(
    _ep_body,
    grid=(GRID,),
    in_specs=[
        pl.BlockSpec((NC, BN, F), lambda i: (0, i, 0)),
        pl.BlockSpec((BN, F), lambda i: (i, 0)),
        pl.BlockSpec((BN, 1), lambda i: (i, 0)),
        pl.BlockSpec((1, F), lambda i: (0, 0)),
        pl.BlockSpec((1, 1), lambda i: (0, 0)),
    ],
    out_specs=pl.BlockSpec((BN, F), lambda i: (i, 0)),
    out_shape=jax.ShapeDtypeStruct((N, F), jnp.float32),
)


def kernel(x, edge_index, W, b, a, u):
    src2 = edge_index[0].astype(jnp.int32).reshape(NW, EPW)
    dst3 = edge_index[1].astype(jnp.int32).reshape(NW, NCHUNK, CH)
    zeros640 = jnp.zeros((640,), jnp.float32)
    ones_ch = jnp.ones((CH,), jnp.float32)
    zrows = jnp.zeros((CH, F), jnp.float32)

    hist = _deg_kernel(dst3, zeros640, ones_ch)
    h = _mm_call(x, W, u.reshape(1, F))
    h2, dis = _scale_call(hist.T, h)
    part = _scatter_kernel(h2, src2, dst3, zrows)
    return _ep_call(part, h2, dis, b.reshape(1, F), a.reshape(1, 1))


# speedup vs baseline: 40.4931x; 1.0125x over previous
import functools

import jax
import jax.numpy as jnp
from jax import lax
from jax.experimental import pallas as pl
from jax.experimental.pallas import tpu as pltpu
from jax.experimental.pallas import tpu_sc as plsc

N = 10000
E = 320000
F = 128
NC = 2
NS = 16
NW = NC * NS
EPW = E // NW
CH = 80
NCHUNK = EPW // CH
NP = NS * 640
BN = 1000
GRID = N // BN

_mesh = plsc.VectorSubcoreMesh(core_axis_name="c", subcore_axis_name="s")


@functools.partial(
    pl.kernel,
    out_type=jax.ShapeDtypeStruct((NC, NP), jnp.float32),
    mesh=_mesh,
    scratch_types=[
        pltpu.VMEM((NCHUNK, CH), jnp.int32),
        pltpu.VMEM((CH,), jnp.float32),
        pltpu.VMEM((640,), jnp.float32),
        pltpu.VMEM_SHARED((NP,), jnp.float32),
        pltpu.SemaphoreType.DMA,
        pltpu.SemaphoreType.DMA,
    ],
)
def _deg_kernel(dst3_hbm, zeros_hbm, ones_hbm, hist_hbm, didx_all, ones_v, zb,
                hist_sp, sem0, sem1):
    c = lax.axis_index("c")
    s = lax.axis_index("s")
    pltpu.sync_copy(zeros_hbm, zb)
    pltpu.sync_copy(ones_hbm, ones_v)
    pltpu.sync_copy(zb, hist_sp.at[pl.ds(s * 640, 640)])
    wid = c * NS + s
    pltpu.sync_copy(dst3_hbm.at[wid], didx_all)
    plsc.subcore_barrier()

    def addchunk(j, sem):
        pltpu.async_copy(ones_v, hist_sp.at[didx_all.at[j]], sem, add=True)

    def drain(sem):
        pltpu.make_async_copy(zeros_hbm.at[pl.ds(0, CH)], ones_v, sem).wait()

    addchunk(0, sem0)

    def pair(p, carry):
        j0 = 2 * p
        addchunk(j0 + 1, sem1)
        drain(sem0)
        addchunk(j0 + 2, sem0)
        drain(sem1)
        return carry

    lax.fori_loop(0, (NCHUNK - 1) // 2, pair, 0)
    drain(sem0)
    plsc.subcore_barrier()
    pltpu.sync_copy(hist_sp.at[pl.ds(s * 640, 640)], zb)
    pltpu.sync_copy(zb, hist_hbm.at[c, pl.ds(s * 640, 640)])


def _mm_body(hist_ref, x_ref, w_ref, u_ref, h2_ref, dis_ref, wsn_ref):
    @pl.when(pl.program_id(0) == 0)
    def _():
        W = w_ref[...]
        u_row = u_ref[...]
        v = lax.dot_general(u_row, W, (((1,), (0,)), ((), ())))
        v = v / (jnp.sqrt(jnp.sum(v * v)) + 1e-12)
        wv = lax.dot_general(v, W, (((1,), (1,)), ((), ())))
        u2 = wv / (jnp.sqrt(jnp.sum(wv * wv)) + 1e-12)
        sigma = jnp.sum(u2 * wv)
        wsn_ref[...] = W / sigma

    ht = hist_ref[...]
    deg = ht[:, 0:1] + ht[:, 1:2] + 1.0
    dis = lax.rsqrt(deg)
    dis_ref[...] = dis
    h = lax.dot_general(x_ref[...], wsn_ref[...], (((1,), (1,)), ((), ())),
                        preferred_element_type=jnp.float32)
    h2_ref[...] = h * dis


_mm_call = pl.pallas_call(
    _mm_body,
    grid=(GRID,),
    in_specs=[
        pl.BlockSpec((BN, NC), lambda i: (i, 0)),
        pl.BlockSpec((BN, F), lambda i: (i, 0)),
        pl.BlockSpec((F, F), lambda i: (0, 0)),
        pl.BlockSpec((1, F), lambda i: (0, 0)),
    ],
    out_specs=[
        pl.BlockSpec((BN, F), lambda i: (i, 0)),
        pl.BlockSpec((BN, 1), lambda i: (i, 0)),
    ],
    out_shape=[
        jax.ShapeDtypeStruct((N, F), jnp.float32),
        jax.ShapeDtypeStruct((N, 1), jnp.float32),
    ],
    scratch_shapes=[pltpu.VMEM((F, F), jnp.float32)],
)


@functools.partial(
    pl.kernel,
    out_type=jax.ShapeDtypeStruct((NC, NP, F), jnp.float32),
    mesh=_mesh,
    scratch_types=[
        pltpu.VMEM((EPW,), jnp.int32),
        pltpu.VMEM((NCHUNK, CH), jnp.int32),
        pltpu.VMEM((CH, F), jnp.float32),
        pltpu.VMEM((CH, F), jnp.float32),
        pltpu.VMEM_SHARED((NP, F), jnp.float32),
        pltpu.SemaphoreType.DMA,
        pltpu.SemaphoreType.DMA,
    ],
)
def _scatter_kernel(h2_hbm, src2_hbm, dst3_hbm, zrows_hbm, part_hbm,
                    sidx_all, didx_all, rows0, rows1, acc_sp, sem0, sem1):
    c = lax.axis_index("c")
    s = lax.axis_index("s")
    wid = c * NS + s
    bufs = (rows0, rows1)
    sems = (sem0, sem1)
    pltpu.sync_copy(zrows_hbm, rows0)
    for k in range(640 // CH):
        pltpu.sync_copy(rows0, acc_sp.at[pl.ds(s * 640 + k * CH, CH)])
    pltpu.sync_copy(src2_hbm.at[wid], sidx_all)
    pltpu.sync_copy(dst3_hbm.at[wid], didx_all)
    plsc.subcore_barrier()

    def gather(j, b):
        pltpu.async_copy(h2_hbm.at[sidx_all.at[pl.ds(j * CH, CH)]],
                         bufs[b], sems[b])

    def gwait(b):
        pltpu.make_async_copy(h2_hbm.at[pl.ds(0, CH)], bufs[b],
                              sems[b]).wait()

    def scat(j, b):
        pltpu.sync_copy(bufs[b], acc_sp.at[didx_all.at[j]], add=True)

    gather(0, 0)

    def pair(p, carry):
        j0 = 2 * p
        gather(j0 + 1, 1)
        gwait(0)
        scat(j0, 0)
        gather(j0 + 2, 0)
        gwait(1)
        scat(j0 + 1, 1)
        return carry

    lax.fori_loop(0, (NCHUNK - 1) // 2, pair, 0)
    gwait(0)
    scat(NCHUNK - 1, 0)
    plsc.subcore_barrier()
    for k in range(640 // CH):
        b = k % 2
        if k >= 2:
            gwait(b)
        pltpu.sync_copy(acc_sp.at[pl.ds(s * 640 + k * CH, CH)], bufs[b])
        pltpu.async_copy(bufs[b], part_hbm.at[c, pl.ds(s * 640 + k * CH, CH)],
                         sems[b])
    gwait(0)
    gwait(1)


def _ep_body(p_ref, h2_ref, dis_ref, b_ref, a_ref, out_ref):
    z = (p_ref[0] + p_ref[1] + h2_ref[...]) * dis_ref[...] + b_ref[...]
    out_ref[...] = jnp.where(z > 0, z, a_ref[0, 0] * z)


_ep_call = pl.pallas_call(
    _ep_body,
    grid=(GRID,),
    in_specs=[
        pl.BlockSpec((NC, BN, F), lambda i: (0, i, 0)),
        pl.BlockSpec((BN, F), lambda i: (i, 0)),
        pl.BlockSpec((BN, 1), lambda i: (i, 0)),
        pl.BlockSpec((1, F), lambda i: (0, 0)),
        pl.BlockSpec((1, 1), lambda i: (0, 0)),
    ],
    out_specs=pl.BlockSpec((BN, F), lambda i: (i, 0)),
    out_shape=jax.ShapeDtypeStruct((N, F), jnp.float32),
)


def kernel(x, edge_index, W, b, a, u):
    src2 = edge_index[0].astype(jnp.int32).reshape(NW, EPW)
    dst3 = edge_index[1].astype(jnp.int32).reshape(NW, NCHUNK, CH)
    zeros640 = jnp.zeros((640,), jnp.float32)
    ones_ch = jnp.ones((CH,), jnp.float32)
    zrows = jnp.zeros((CH, F), jnp.float32)

    hist = _deg_kernel(dst3, zeros640, ones_ch)
    h2, dis = _mm_call(hist.T, x, W, u.reshape(1, F))
    part = _scatter_kernel(h2, src2, dst3, zrows)
    return _ep_call(part, h2, dis, b.reshape(1, F), a.reshape(1, 1))
